# baseline (speedup 1.0000x reference)
import functools

import numpy as np

import jax
import jax.numpy as jnp
from jax import lax
from jax.experimental import pallas as pl
from jax.experimental.pallas import tpu as pltpu
from jax.experimental.pallas import tpu_sc as plsc

_LANES = 16


def _proj_tc(edge_attr, Wt, b2d):
    E, R = edge_attr.shape
    H = Wt.shape[1]
    BE = 2000
    assert E % BE == 0

    def body(ea_ref, wt_ref, b_ref, out_ref):
        out_ref[...] = (
            jnp.dot(ea_ref[...], wt_ref[...], preferred_element_type=jnp.float32)
            + b_ref[...]
        ).astype(jnp.bfloat16)

    return pl.pallas_call(
        body,
        grid=(E // BE,),
        in_specs=[
            pl.BlockSpec((BE, R), lambda i: (i, 0)),
            pl.BlockSpec((R, H), lambda i: (0, 0)),
            pl.BlockSpec((1, H), lambda i: (0, 0)),
        ],
        out_specs=pl.BlockSpec((BE, H), lambda i: (i, 0)),
        out_shape=jax.ShapeDtypeStruct((E, H), jnp.bfloat16),
    )(edge_attr, Wt, b2d)


def _sc_combine(src, dst, proj, x):
    E = src.shape[0]
    V, H = x.shape
    info = plsc.get_sparse_core_info()
    NC, NS = info.num_cores, info.num_subcores
    NW = NC * NS
    assert E % NW == 0
    epw = E // NW
    B = 80
    assert epw % B == 0
    nblk = epw // B
    assert nblk % 2 == 1
    HC2 = H // (2 * _LANES)

    mesh = plsc.VectorSubcoreMesh(core_axis_name="c", subcore_axis_name="s")

    @functools.partial(
        pl.kernel,
        mesh=mesh,
        out_type=jax.ShapeDtypeStruct((E, H), jnp.float32),
        scratch_types=(
            [pltpu.VMEM((B,), jnp.int32) for _ in range(4)]
            + [pltpu.VMEM((B, H), jnp.float32) for _ in range(4)]
            + [pltpu.VMEM((B, H // 2), jnp.int32) for _ in range(2)]
            + [pltpu.VMEM((B, H), jnp.float32) for _ in range(2)]
            + [pltpu.SemaphoreType.DMA for _ in range(12)]
        ),
    )
    def k(src_hbm, dst_hbm, proj_hbm, x_hbm, out_hbm,
          is0, is1, id0, id1, xi0, xi1, xj0, xj1, pr0, pr1, ob0, ob1,
          sis0, sis1, sid0, sid1, sgi0, sgi1, sgj0, sgj1, spr0, spr1,
          sou0, sou1):
        idx_s, idx_d = (is0, is1), (id0, id1)
        xi, xj, pr, ob = (xi0, xi1), (xj0, xj1), (pr0, pr1), (ob0, ob1)
        sis, sid = (sis0, sis1), (sid0, sid1)
        sgi, sgj, spr, sou = (sgi0, sgi1), (sgj0, sgj1), (spr0, spr1), (sou0, sou1)

        wid = lax.axis_index("s") * NC + lax.axis_index("c")
        wbase = wid * epw

        def issue_idx(g, p):
            base = wbase + g * B
            pltpu.async_copy(src_hbm.at[pl.ds(base, B)], idx_s[p], sis[p])
            pltpu.async_copy(dst_hbm.at[pl.ds(base, B)], idx_d[p], sid[p])

        def wait_idx(p):
            pltpu.make_async_copy(src_hbm.at[pl.ds(0, B)], idx_s[p], sis[p]).wait()
            pltpu.make_async_copy(dst_hbm.at[pl.ds(0, B)], idx_d[p], sid[p]).wait()

        def issue_fetch(g, p):
            base = wbase + g * B
            pltpu.async_copy(x_hbm.at[idx_s[p]], xi[p], sgi[p])
            pltpu.async_copy(x_hbm.at[idx_d[p]], xj[p], sgj[p])
            pltpu.async_copy(proj_hbm.at[pl.ds(base, B), :], pr[p], spr[p])

        def wait_fetch(p):
            pltpu.make_async_copy(x_hbm.at[idx_s[p]], xi[p], sgi[p]).wait()
            pltpu.make_async_copy(x_hbm.at[idx_d[p]], xj[p], sgj[p]).wait()
            pltpu.make_async_copy(
                proj_hbm.at[pl.ds(0, B), :], pr[p], spr[p]).wait()

        def issue_out(g, p):
            base = wbase + g * B
            pltpu.async_copy(ob[p], out_hbm.at[pl.ds(base, B), :], sou[p])

        def wait_out(p):
            pltpu.make_async_copy(ob[p], out_hbm.at[pl.ds(0, B), :], sou[p]).wait()

        def combine(p):
            xi_p, xj_p, pr_p, ob_p = xi[p], xj[p], pr[p], ob[p]

            def edge(e, c2):
                for c in range(HC2):
                    pi = pr_p[e, pl.ds(c * _LANES, _LANES)]
                    pa = lax.bitcast_convert_type(
                        jnp.left_shift(pi, 16), jnp.float32)
                    pb = lax.bitcast_convert_type(
                        jnp.bitwise_and(pi, jnp.int32(-65536)), jnp.float32)
                    s0 = pl.ds(c * 2 * _LANES, _LANES)
                    s1 = pl.ds(c * 2 * _LANES + _LANES, _LANES)
                    ob_p[e, s0] = (xi_p[e, s0] + xj_p[e, s0]) * pa
                    ob_p[e, s1] = (xi_p[e, s1] + xj_p[e, s1]) * pb
                return c2

            lax.fori_loop(0, B, edge, 0)

        def step(g, p):
            wait_fetch(p)
            wait_idx(1 - p)
            issue_fetch(g + 1, 1 - p)
            pl.when(g + 2 <= nblk - 1)(lambda: issue_idx(g + 2, p))
            pl.when(g >= 2)(lambda: wait_out(p))
            combine(p)
            issue_out(g, p)

        issue_idx(0, 0)
        wait_idx(0)
        issue_fetch(0, 0)
        issue_idx(1, 1)

        def pair(i, carry):
            step(2 * i, 0)
            step(2 * i + 1, 1)
            return carry

        lax.fori_loop(0, (nblk - 1) // 2, pair, 0)

        g_last = nblk - 1
        wait_fetch(0)
        wait_out(0)
        combine(0)
        issue_out(g_last, 0)
        wait_out(1)
        wait_out(0)

    return k(src, dst, proj, x)


def _interleave_perm(H):
    m = np.arange(H)
    return (m // 32) * 32 + 16 * (m % 2) + (m % 32) // 2


def kernel(edge_index, edge_attr, x, W, b):
    src = edge_index[0].astype(jnp.int32)
    dst = edge_index[1].astype(jnp.int32)
    H = W.shape[0]
    perm = jnp.asarray(_interleave_perm(H))
    Wt_p = W.T[:, perm]
    b_p = b[perm].reshape(1, H)
    proj = _proj_tc(edge_attr, Wt_p, b_p)
    proj_i32 = jax.lax.bitcast_convert_type(
        proj.reshape(proj.shape[0], H // 2, 2), jnp.int32)
    return _sc_combine(src, dst, proj_i32, x)


# speedup vs baseline: 2.5547x; 2.5547x over previous
import functools

import jax
import jax.numpy as jnp
from jax import lax
from jax.experimental import pallas as pl
from jax.experimental.pallas import tpu as pltpu
from jax.experimental.pallas import tpu_sc as plsc

_LANES = 16


def _round_bf16_bits(f32_arr):
    u = lax.bitcast_convert_type(f32_arr, jnp.uint32)
    return u + jnp.uint32(0x7FFF) + ((u >> 16) & jnp.uint32(1))


def _proj_tc_packed(ea32, w2a, w2b, b2d):
    E2, R2 = ea32.shape
    H = w2a.shape[1]
    BE = 1000
    assert E2 % BE == 0

    def body(ea_ref, wa_ref, wb_ref, b_ref, out_ref):
        pa = jnp.dot(ea_ref[...], wa_ref[...],
                     preferred_element_type=jnp.float32) + b_ref[...]
        pb = jnp.dot(ea_ref[...], wb_ref[...],
                     preferred_element_type=jnp.float32) + b_ref[...]
        lo = _round_bf16_bits(pa) >> 16
        hi = _round_bf16_bits(pb) & jnp.uint32(0xFFFF0000)
        out_ref[...] = hi | lo

    return pl.pallas_call(
        body,
        grid=(E2 // BE,),
        in_specs=[
            pl.BlockSpec((BE, R2), lambda i: (i, 0)),
            pl.BlockSpec((R2, H), lambda i: (0, 0)),
            pl.BlockSpec((R2, H), lambda i: (0, 0)),
            pl.BlockSpec((1, H), lambda i: (0, 0)),
        ],
        out_specs=pl.BlockSpec((BE, H), lambda i: (i, 0)),
        out_shape=jax.ShapeDtypeStruct((E2, H), jnp.uint32),
    )(ea32, w2a, w2b, b2d)


def _sc_combine(src, dst, projp, x):
    E = src.shape[0]
    V, H = x.shape
    info = plsc.get_sparse_core_info()
    NC, NS = info.num_cores, info.num_subcores
    NW = NC * NS
    assert E % NW == 0
    epw = E // NW
    B = 80
    assert epw % B == 0 and B % 16 == 0
    nblk = epw // B
    assert nblk % 2 == 1
    HC = H // _LANES

    mesh = plsc.VectorSubcoreMesh(core_axis_name="c", subcore_axis_name="s")

    @functools.partial(
        pl.kernel,
        mesh=mesh,
        out_type=jax.ShapeDtypeStruct((E, H), jnp.float32),
        scratch_types=(
            [pltpu.VMEM((B,), jnp.int32) for _ in range(4)]
            + [pltpu.VMEM((B, H), jnp.float32) for _ in range(4)]
            + [pltpu.VMEM((B // 2, H), jnp.uint32) for _ in range(2)]
            + [pltpu.VMEM((B, H), jnp.float32) for _ in range(2)]
            + [pltpu.SemaphoreType.DMA for _ in range(12)]
        ),
    )
    def k(src_hbm, dst_hbm, proj_hbm, x_hbm, out_hbm,
          is0, is1, id0, id1, xi0, xi1, xj0, xj1, pr0, pr1, ob0, ob1,
          sis0, sis1, sid0, sid1, sgi0, sgi1, sgj0, sgj1, spr0, spr1,
          sou0, sou1):
        idx_s, idx_d = (is0, is1), (id0, id1)
        xi, xj, pr, ob = (xi0, xi1), (xj0, xj1), (pr0, pr1), (ob0, ob1)
        sis, sid = (sis0, sis1), (sid0, sid1)
        sgi, sgj, spr, sou = (sgi0, sgi1), (sgj0, sgj1), (spr0, spr1), (sou0, sou1)

        wid = lax.axis_index("s") * NC + lax.axis_index("c")
        wbase = wid * epw

        def issue_idx(g, p):
            base = wbase + g * B
            pltpu.async_copy(src_hbm.at[pl.ds(base, B)], idx_s[p], sis[p])
            pltpu.async_copy(dst_hbm.at[pl.ds(base, B)], idx_d[p], sid[p])

        def wait_idx(p):
            pltpu.make_async_copy(src_hbm.at[pl.ds(0, B)], idx_s[p], sis[p]).wait()
            pltpu.make_async_copy(dst_hbm.at[pl.ds(0, B)], idx_d[p], sid[p]).wait()

        def issue_fetch(g, p):
            base = wbase + g * B
            pltpu.async_copy(x_hbm.at[idx_s[p]], xi[p], sgi[p])
            pltpu.async_copy(x_hbm.at[idx_d[p]], xj[p], sgj[p])
            pltpu.async_copy(
                proj_hbm.at[pl.ds(pl.multiple_of(base // 2, 8), B // 2), :],
                pr[p], spr[p])

        def wait_fetch(p):
            pltpu.make_async_copy(x_hbm.at[idx_s[p]], xi[p], sgi[p]).wait()
            pltpu.make_async_copy(x_hbm.at[idx_d[p]], xj[p], sgj[p]).wait()
            pltpu.make_async_copy(
                proj_hbm.at[pl.ds(0, B // 2), :], pr[p], spr[p]).wait()

        def issue_out(g, p):
            base = wbase + g * B
            pltpu.async_copy(ob[p], out_hbm.at[pl.ds(base, B), :], sou[p])

        def wait_out(p):
            pltpu.make_async_copy(ob[p], out_hbm.at[pl.ds(0, B), :], sou[p]).wait()

        def combine(p):
            xi_p, xj_p, pr_p, ob_p = xi[p], xj[p], pr[p], ob[p]

            def pair_rows(rp, c2):
                ea = rp * 2
                eb = rp * 2 + 1
                for c in range(HC):
                    s = pl.ds(c * _LANES, _LANES)
                    pi = pr_p[rp, s]
                    pa = lax.bitcast_convert_type(
                        jnp.left_shift(pi, 16), jnp.float32)
                    pb = lax.bitcast_convert_type(
                        jnp.bitwise_and(pi, jnp.uint32(0xFFFF0000)),
                        jnp.float32)
                    ob_p[ea, s] = (xi_p[ea, s] + xj_p[ea, s]) * pa
                    ob_p[eb, s] = (xi_p[eb, s] + xj_p[eb, s]) * pb
                return c2

            lax.fori_loop(0, B // 2, pair_rows, 0)

        def step(g, p):
            wait_fetch(p)
            wait_idx(1 - p)
            issue_fetch(g + 1, 1 - p)
            pl.when(g + 2 <= nblk - 1)(lambda: issue_idx(g + 2, p))
            pl.when(g >= 2)(lambda: wait_out(p))
            combine(p)
            issue_out(g, p)

        issue_idx(0, 0)
        wait_idx(0)
        issue_fetch(0, 0)
        issue_idx(1, 1)

        def pair(i, carry):
            step(2 * i, 0)
            step(2 * i + 1, 1)
            return carry

        lax.fori_loop(0, (nblk - 1) // 2, pair, 0)

        g_last = nblk - 1
        wait_fetch(0)
        wait_out(0)
        combine(0)
        issue_out(g_last, 0)
        wait_out(1)
        wait_out(0)

    return k(src, dst, projp, x)


def kernel(edge_index, edge_attr, x, W, b):
    src = edge_index[0].astype(jnp.int32)
    dst = edge_index[1].astype(jnp.int32)
    H, R = W.shape
    E = edge_attr.shape[0]
    Wt = W.T
    zeros = jnp.zeros_like(Wt)
    w2a = jnp.concatenate([Wt, zeros], axis=0)
    w2b = jnp.concatenate([zeros, Wt], axis=0)
    ea32 = edge_attr.reshape(E // 2, 2 * R)
    projp = _proj_tc_packed(ea32, w2a, w2b, b.reshape(1, H))
    return _sc_combine(src, dst, projp, x)


# speedup vs baseline: 2.9284x; 1.1463x over previous
import functools

import jax
import jax.numpy as jnp
from jax import lax
from jax.experimental import pallas as pl
from jax.experimental.pallas import tpu as pltpu
from jax.experimental.pallas import tpu_sc as plsc

_LANES = 16


def _round_bf16_bits(f32_arr):
    u = lax.bitcast_convert_type(f32_arr, jnp.uint32)
    return u + jnp.uint32(0x7FFF) + ((u >> 16) & jnp.uint32(1))


def _proj_tc_packed(ea32, w2a, w2b, b2d):
    E2, R2 = ea32.shape
    H = w2a.shape[1]
    BE = 4000
    assert E2 % BE == 0

    def body(ea_ref, wa_ref, wb_ref, b_ref, out_ref):
        pa = jnp.dot(ea_ref[...], wa_ref[...],
                     preferred_element_type=jnp.float32) + b_ref[...]
        pb = jnp.dot(ea_ref[...], wb_ref[...],
                     preferred_element_type=jnp.float32) + b_ref[...]
        lo = _round_bf16_bits(pa) >> 16
        hi = _round_bf16_bits(pb) & jnp.uint32(0xFFFF0000)
        out_ref[...] = hi | lo

    return pl.pallas_call(
        body,
        grid=(E2 // BE,),
        in_specs=[
            pl.BlockSpec((BE, R2), lambda i: (i, 0)),
            pl.BlockSpec((R2, H), lambda i: (0, 0)),
            pl.BlockSpec((R2, H), lambda i: (0, 0)),
            pl.BlockSpec((1, H), lambda i: (0, 0)),
        ],
        out_specs=pl.BlockSpec((BE, H), lambda i: (i, 0)),
        out_shape=jax.ShapeDtypeStruct((E2, H), jnp.uint32),
    )(ea32, w2a, w2b, b2d)


def _sc_combine(src, dst, projp, x):
    E = src.shape[0]
    V, H = x.shape
    info = plsc.get_sparse_core_info()
    NC, NS = info.num_cores, info.num_subcores
    NW = NC * NS
    assert E % NW == 0
    epw = E // NW
    B = 80
    assert epw % B == 0 and B % 16 == 0
    nblk = epw // B
    assert nblk % 2 == 1
    HC = H // _LANES

    mesh = plsc.VectorSubcoreMesh(core_axis_name="c", subcore_axis_name="s")

    @functools.partial(
        pl.kernel,
        mesh=mesh,
        out_type=jax.ShapeDtypeStruct((E, H), jnp.float32),
        scratch_types=(
            [pltpu.VMEM((B,), jnp.int32) for _ in range(4)]
            + [pltpu.VMEM((B, H), jnp.float32) for _ in range(4)]
            + [pltpu.VMEM((B // 2, H), jnp.uint32) for _ in range(2)]
            + [pltpu.VMEM((B, H), jnp.float32) for _ in range(2)]
            + [pltpu.SemaphoreType.DMA for _ in range(12)]
        ),
    )
    def k(src_hbm, dst_hbm, proj_hbm, x_hbm, out_hbm,
          is0, is1, id0, id1, xi0, xi1, xj0, xj1, pr0, pr1, ob0, ob1,
          sis0, sis1, sid0, sid1, sgi0, sgi1, sgj0, sgj1, spr0, spr1,
          sou0, sou1):
        idx_s, idx_d = (is0, is1), (id0, id1)
        xi, xj, pr, ob = (xi0, xi1), (xj0, xj1), (pr0, pr1), (ob0, ob1)
        sis, sid = (sis0, sis1), (sid0, sid1)
        sgi, sgj, spr, sou = (sgi0, sgi1), (sgj0, sgj1), (spr0, spr1), (sou0, sou1)

        wid = lax.axis_index("s") * NC + lax.axis_index("c")
        wbase = wid * epw

        def issue_idx(g, p):
            base = wbase + g * B
            pltpu.async_copy(src_hbm.at[pl.ds(base, B)], idx_s[p], sis[p])
            pltpu.async_copy(dst_hbm.at[pl.ds(base, B)], idx_d[p], sid[p])

        def wait_idx(p):
            pltpu.make_async_copy(src_hbm.at[pl.ds(0, B)], idx_s[p], sis[p]).wait()
            pltpu.make_async_copy(dst_hbm.at[pl.ds(0, B)], idx_d[p], sid[p]).wait()

        def issue_fetch(g, p):
            base = wbase + g * B
            pltpu.async_copy(x_hbm.at[idx_s[p]], xi[p], sgi[p])
            pltpu.async_copy(x_hbm.at[idx_d[p]], xj[p], sgj[p])
            pltpu.async_copy(
                proj_hbm.at[pl.ds(pl.multiple_of(base // 2, 8), B // 2), :],
                pr[p], spr[p])

        def wait_fetch(p):
            pltpu.make_async_copy(x_hbm.at[idx_s[p]], xi[p], sgi[p]).wait()
            pltpu.make_async_copy(x_hbm.at[idx_d[p]], xj[p], sgj[p]).wait()
            pltpu.make_async_copy(
                proj_hbm.at[pl.ds(0, B // 2), :], pr[p], spr[p]).wait()

        def issue_out(g, p):
            base = wbase + g * B
            pltpu.async_copy(ob[p], out_hbm.at[pl.ds(base, B), :], sou[p])

        def wait_out(p):
            pltpu.make_async_copy(ob[p], out_hbm.at[pl.ds(0, B), :], sou[p]).wait()

        def combine(p):
            xi_p, xj_p, pr_p, ob_p = xi[p], xj[p], pr[p], ob[p]

            def pair_rows(rp, c2):
                ea = rp * 2
                eb = rp * 2 + 1
                for c in range(HC):
                    s = pl.ds(c * _LANES, _LANES)
                    pi = pr_p[rp, s]
                    pa = lax.bitcast_convert_type(
                        jnp.left_shift(pi, 16), jnp.float32)
                    pb = lax.bitcast_convert_type(
                        jnp.bitwise_and(pi, jnp.uint32(0xFFFF0000)),
                        jnp.float32)
                    ob_p[ea, s] = (xi_p[ea, s] + xj_p[ea, s]) * pa
                    ob_p[eb, s] = (xi_p[eb, s] + xj_p[eb, s]) * pb
                return c2

            lax.fori_loop(0, B // 2, pair_rows, 0)

        def step(g, p):
            wait_fetch(p)
            wait_idx(1 - p)
            issue_fetch(g + 1, 1 - p)
            pl.when(g + 2 <= nblk - 1)(lambda: issue_idx(g + 2, p))
            pl.when(g >= 2)(lambda: wait_out(p))
            combine(p)
            issue_out(g, p)

        issue_idx(0, 0)
        wait_idx(0)
        issue_fetch(0, 0)
        issue_idx(1, 1)

        def pair(i, carry):
            step(2 * i, 0)
            step(2 * i + 1, 1)
            return carry

        lax.fori_loop(0, (nblk - 1) // 2, pair, 0)

        g_last = nblk - 1
        wait_fetch(0)
        wait_out(0)
        combine(0)
        issue_out(g_last, 0)
        wait_out(1)
        wait_out(0)

    return k(src, dst, projp, x)


def kernel(edge_index, edge_attr, x, W, b):
    src = edge_index[0].astype(jnp.int32)
    dst = edge_index[1].astype(jnp.int32)
    H, R = W.shape
    E = edge_attr.shape[0]
    Wt = W.T
    zeros = jnp.zeros_like(Wt)
    w2a = jnp.concatenate([Wt, zeros], axis=0)
    w2b = jnp.concatenate([zeros, Wt], axis=0)
    ea32 = edge_attr.reshape(E // 2, 2 * R)
    projp = _proj_tc_packed(ea32, w2a, w2b, b.reshape(1, H))
    return _sc_combine(src, dst, projp, x)


# speedup vs baseline: 2.9967x; 1.0233x over previous
import functools

import jax
import jax.numpy as jnp
from jax import lax
from jax.experimental import pallas as pl
from jax.experimental.pallas import tpu as pltpu
from jax.experimental.pallas import tpu_sc as plsc

_LANES = 16


def _round_bf16_bits(f32_arr):
    u = lax.bitcast_convert_type(f32_arr, jnp.uint32)
    return u + jnp.uint32(0x7FFF) + ((u >> 16) & jnp.uint32(1))


def _proj_tc_packed(ea32, w2a, w2b, b2d):
    E2, R2 = ea32.shape
    H = w2a.shape[1]
    BE = 8000
    assert E2 % BE == 0

    def body(ea_ref, wa_ref, wb_ref, b_ref, out_ref):
        pa = jnp.dot(ea_ref[...], wa_ref[...],
                     preferred_element_type=jnp.float32) + b_ref[...]
        pb = jnp.dot(ea_ref[...], wb_ref[...],
                     preferred_element_type=jnp.float32) + b_ref[...]
        lo = _round_bf16_bits(pa) >> 16
        hi = _round_bf16_bits(pb) & jnp.uint32(0xFFFF0000)
        out_ref[...] = hi | lo

    return pl.pallas_call(
        body,
        grid=(E2 // BE,),
        in_specs=[
            pl.BlockSpec((BE, R2), lambda i: (i, 0)),
            pl.BlockSpec((R2, H), lambda i: (0, 0)),
            pl.BlockSpec((R2, H), lambda i: (0, 0)),
            pl.BlockSpec((1, H), lambda i: (0, 0)),
        ],
        out_specs=pl.BlockSpec((BE, H), lambda i: (i, 0)),
        out_shape=jax.ShapeDtypeStruct((E2, H), jnp.uint32),
    )(ea32, w2a, w2b, b2d)


def _sc_combine(src, dst, projp, x):
    E = src.shape[0]
    V, H = x.shape
    info = plsc.get_sparse_core_info()
    NC, NS = info.num_cores, info.num_subcores
    NW = NC * NS
    assert E % NW == 0
    epw = E // NW
    B = 80
    assert epw % B == 0 and B % 16 == 0
    nblk = epw // B
    assert nblk % 2 == 1
    HC = H // _LANES

    mesh = plsc.VectorSubcoreMesh(core_axis_name="c", subcore_axis_name="s")

    @functools.partial(
        pl.kernel,
        mesh=mesh,
        out_type=jax.ShapeDtypeStruct((E, H), jnp.float32),
        scratch_types=(
            [pltpu.VMEM((B,), jnp.int32) for _ in range(4)]
            + [pltpu.VMEM((B, H), jnp.float32) for _ in range(4)]
            + [pltpu.VMEM((B // 2, H), jnp.uint32) for _ in range(2)]
            + [pltpu.VMEM((B, H), jnp.float32) for _ in range(2)]
            + [pltpu.SemaphoreType.DMA for _ in range(12)]
        ),
    )
    def k(src_hbm, dst_hbm, proj_hbm, x_hbm, out_hbm,
          is0, is1, id0, id1, xi0, xi1, xj0, xj1, pr0, pr1, ob0, ob1,
          sis0, sis1, sid0, sid1, sgi0, sgi1, sgj0, sgj1, spr0, spr1,
          sou0, sou1):
        idx_s, idx_d = (is0, is1), (id0, id1)
        xi, xj, pr, ob = (xi0, xi1), (xj0, xj1), (pr0, pr1), (ob0, ob1)
        sis, sid = (sis0, sis1), (sid0, sid1)
        sgi, sgj, spr, sou = (sgi0, sgi1), (sgj0, sgj1), (spr0, spr1), (sou0, sou1)

        wid = lax.axis_index("s") * NC + lax.axis_index("c")
        wbase = wid * epw

        def issue_idx(g, p):
            base = wbase + g * B
            pltpu.async_copy(src_hbm.at[pl.ds(base, B)], idx_s[p], sis[p])
            pltpu.async_copy(dst_hbm.at[pl.ds(base, B)], idx_d[p], sid[p])

        def wait_idx(p):
            pltpu.make_async_copy(src_hbm.at[pl.ds(0, B)], idx_s[p], sis[p]).wait()
            pltpu.make_async_copy(dst_hbm.at[pl.ds(0, B)], idx_d[p], sid[p]).wait()

        def issue_fetch(g, p):
            base = wbase + g * B
            pltpu.async_copy(x_hbm.at[idx_s[p]], xi[p], sgi[p])
            pltpu.async_copy(x_hbm.at[idx_d[p]], xj[p], sgj[p])
            pltpu.async_copy(
                proj_hbm.at[pl.ds(pl.multiple_of(base // 2, 8), B // 2), :],
                pr[p], spr[p])

        def wait_fetch(p):
            pltpu.make_async_copy(x_hbm.at[idx_s[p]], xi[p], sgi[p]).wait()
            pltpu.make_async_copy(x_hbm.at[idx_d[p]], xj[p], sgj[p]).wait()
            pltpu.make_async_copy(
                proj_hbm.at[pl.ds(0, B // 2), :], pr[p], spr[p]).wait()

        def issue_out(g, p):
            base = wbase + g * B
            pltpu.async_copy(ob[p], out_hbm.at[pl.ds(base, B), :], sou[p])

        def wait_out(p):
            pltpu.make_async_copy(ob[p], out_hbm.at[pl.ds(0, B), :], sou[p]).wait()

        def combine(p):
            xi_p, xj_p, pr_p, ob_p = xi[p], xj[p], pr[p], ob[p]

            def pair_rows(rp, c2):
                ea = rp * 2
                eb = rp * 2 + 1
                for c in range(HC):
                    s = pl.ds(c * _LANES, _LANES)
                    pi = pr_p[rp, s]
                    pa = lax.bitcast_convert_type(
                        jnp.left_shift(pi, 16), jnp.float32)
                    pb = lax.bitcast_convert_type(
                        jnp.bitwise_and(pi, jnp.uint32(0xFFFF0000)),
                        jnp.float32)
                    ob_p[ea, s] = (xi_p[ea, s] + xj_p[ea, s]) * pa
                    ob_p[eb, s] = (xi_p[eb, s] + xj_p[eb, s]) * pb
                return c2

            lax.fori_loop(0, B // 2, pair_rows, 0)

        def step(g, p):
            wait_fetch(p)
            wait_idx(1 - p)
            issue_fetch(g + 1, 1 - p)
            pl.when(g + 2 <= nblk - 1)(lambda: issue_idx(g + 2, p))
            pl.when(g >= 2)(lambda: wait_out(p))
            combine(p)
            issue_out(g, p)

        issue_idx(0, 0)
        wait_idx(0)
        issue_fetch(0, 0)
        issue_idx(1, 1)

        def pair(i, carry):
            step(2 * i, 0)
            step(2 * i + 1, 1)
            return carry

        lax.fori_loop(0, (nblk - 1) // 2, pair, 0)

        g_last = nblk - 1
        wait_fetch(0)
        wait_out(0)
        combine(0)
        issue_out(g_last, 0)
        wait_out(1)
        wait_out(0)

    return k(src, dst, projp, x)


def kernel(edge_index, edge_attr, x, W, b):
    src = edge_index[0].astype(jnp.int32)
    dst = edge_index[1].astype(jnp.int32)
    H, R = W.shape
    E = edge_attr.shape[0]
    Wt = W.T
    zeros = jnp.zeros_like(Wt)
    w2a = jnp.concatenate([Wt, zeros], axis=0)
    w2b = jnp.concatenate([zeros, Wt], axis=0)
    ea32 = edge_attr.reshape(E // 2, 2 * R)
    projp = _proj_tc_packed(ea32, w2a, w2b, b.reshape(1, H))
    return _sc_combine(src, dst, projp, x)


# speedup vs baseline: 3.0179x; 1.0071x over previous
import functools

import jax
import jax.numpy as jnp
from jax import lax
from jax.experimental import pallas as pl
from jax.experimental.pallas import tpu as pltpu
from jax.experimental.pallas import tpu_sc as plsc

_LANES = 16


def _round_bf16_bits(f32_arr):
    u = lax.bitcast_convert_type(f32_arr, jnp.uint32)
    return u + jnp.uint32(0x7FFF) + ((u >> 16) & jnp.uint32(1))


def _proj_tc_packed(ea32, w2a, w2b, b2d):
    E2, R2 = ea32.shape
    H = w2a.shape[1]
    BE = 16000
    assert E2 % BE == 0

    def body(ea_ref, wa_ref, wb_ref, b_ref, out_ref):
        pa = jnp.dot(ea_ref[...], wa_ref[...],
                     preferred_element_type=jnp.float32) + b_ref[...]
        pb = jnp.dot(ea_ref[...], wb_ref[...],
                     preferred_element_type=jnp.float32) + b_ref[...]
        lo = _round_bf16_bits(pa) >> 16
        hi = _round_bf16_bits(pb) & jnp.uint32(0xFFFF0000)
        out_ref[...] = hi | lo

    return pl.pallas_call(
        body,
        grid=(E2 // BE,),
        in_specs=[
            pl.BlockSpec((BE, R2), lambda i: (i, 0)),
            pl.BlockSpec((R2, H), lambda i: (0, 0)),
            pl.BlockSpec((R2, H), lambda i: (0, 0)),
            pl.BlockSpec((1, H), lambda i: (0, 0)),
        ],
        out_specs=pl.BlockSpec((BE, H), lambda i: (i, 0)),
        out_shape=jax.ShapeDtypeStruct((E2, H), jnp.uint32),
    )(ea32, w2a, w2b, b2d)


def _sc_combine(src, dst, projp, x):
    E = src.shape[0]
    V, H = x.shape
    info = plsc.get_sparse_core_info()
    NC, NS = info.num_cores, info.num_subcores
    NW = NC * NS
    assert E % NW == 0
    epw = E // NW
    B = 80
    assert epw % B == 0 and B % 16 == 0
    nblk = epw // B
    assert nblk % 2 == 1
    HC = H // _LANES

    mesh = plsc.VectorSubcoreMesh(core_axis_name="c", subcore_axis_name="s")

    @functools.partial(
        pl.kernel,
        mesh=mesh,
        out_type=jax.ShapeDtypeStruct((E, H), jnp.float32),
        scratch_types=(
            [pltpu.VMEM((B,), jnp.int32) for _ in range(4)]
            + [pltpu.VMEM((B, H), jnp.float32) for _ in range(4)]
            + [pltpu.VMEM((B // 2, H), jnp.uint32) for _ in range(2)]
            + [pltpu.VMEM((B, H), jnp.float32) for _ in range(2)]
            + [pltpu.SemaphoreType.DMA for _ in range(12)]
        ),
    )
    def k(src_hbm, dst_hbm, proj_hbm, x_hbm, out_hbm,
          is0, is1, id0, id1, xi0, xi1, xj0, xj1, pr0, pr1, ob0, ob1,
          sis0, sis1, sid0, sid1, sgi0, sgi1, sgj0, sgj1, spr0, spr1,
          sou0, sou1):
        idx_s, idx_d = (is0, is1), (id0, id1)
        xi, xj, pr, ob = (xi0, xi1), (xj0, xj1), (pr0, pr1), (ob0, ob1)
        sis, sid = (sis0, sis1), (sid0, sid1)
        sgi, sgj, spr, sou = (sgi0, sgi1), (sgj0, sgj1), (spr0, spr1), (sou0, sou1)

        wid = lax.axis_index("s") * NC + lax.axis_index("c")
        wbase = wid * epw

        def issue_idx(g, p):
            base = wbase + g * B
            pltpu.async_copy(src_hbm.at[pl.ds(base, B)], idx_s[p], sis[p])
            pltpu.async_copy(dst_hbm.at[pl.ds(base, B)], idx_d[p], sid[p])

        def wait_idx(p):
            pltpu.make_async_copy(src_hbm.at[pl.ds(0, B)], idx_s[p], sis[p]).wait()
            pltpu.make_async_copy(dst_hbm.at[pl.ds(0, B)], idx_d[p], sid[p]).wait()

        def issue_fetch(g, p):
            base = wbase + g * B
            pltpu.async_copy(x_hbm.at[idx_s[p]], xi[p], sgi[p])
            pltpu.async_copy(x_hbm.at[idx_d[p]], xj[p], sgj[p])
            pltpu.async_copy(
                proj_hbm.at[pl.ds(pl.multiple_of(base // 2, 8), B // 2), :],
                pr[p], spr[p])

        def wait_fetch(p):
            pltpu.make_async_copy(x_hbm.at[idx_s[p]], xi[p], sgi[p]).wait()
            pltpu.make_async_copy(x_hbm.at[idx_d[p]], xj[p], sgj[p]).wait()
            pltpu.make_async_copy(
                proj_hbm.at[pl.ds(0, B // 2), :], pr[p], spr[p]).wait()

        def issue_out(g, p):
            base = wbase + g * B
            pltpu.async_copy(ob[p], out_hbm.at[pl.ds(base, B), :], sou[p])

        def wait_out(p):
            pltpu.make_async_copy(ob[p], out_hbm.at[pl.ds(0, B), :], sou[p]).wait()

        def combine(p):
            xi_p, xj_p, pr_p, ob_p = xi[p], xj[p], pr[p], ob[p]

            def pair_rows(rp, c2):
                ea = rp * 2
                eb = rp * 2 + 1
                for c in range(HC):
                    s = pl.ds(c * _LANES, _LANES)
                    pi = pr_p[rp, s]
                    pa = lax.bitcast_convert_type(
                        jnp.left_shift(pi, 16), jnp.float32)
                    pb = lax.bitcast_convert_type(
                        jnp.bitwise_and(pi, jnp.uint32(0xFFFF0000)),
                        jnp.float32)
                    ob_p[ea, s] = (xi_p[ea, s] + xj_p[ea, s]) * pa
                    ob_p[eb, s] = (xi_p[eb, s] + xj_p[eb, s]) * pb
                return c2

            lax.fori_loop(0, B // 2, pair_rows, 0)

        def step(g, p):
            wait_fetch(p)
            wait_idx(1 - p)
            issue_fetch(g + 1, 1 - p)
            pl.when(g + 2 <= nblk - 1)(lambda: issue_idx(g + 2, p))
            pl.when(g >= 2)(lambda: wait_out(p))
            combine(p)
            issue_out(g, p)

        issue_idx(0, 0)
        wait_idx(0)
        issue_fetch(0, 0)
        issue_idx(1, 1)

        def pair(i, carry):
            step(2 * i, 0)
            step(2 * i + 1, 1)
            return carry

        lax.fori_loop(0, (nblk - 1) // 2, pair, 0)

        g_last = nblk - 1
        wait_fetch(0)
        wait_out(0)
        combine(0)
        issue_out(g_last, 0)
        wait_out(1)
        wait_out(0)

    return k(src, dst, projp, x)


def kernel(edge_index, edge_attr, x, W, b):
    src = edge_index[0].astype(jnp.int32)
    dst = edge_index[1].astype(jnp.int32)
    H, R = W.shape
    E = edge_attr.shape[0]
    Wt = W.T
    zeros = jnp.zeros_like(Wt)
    w2a = jnp.concatenate([Wt, zeros], axis=0)
    w2b = jnp.concatenate([zeros, Wt], axis=0)
    ea32 = edge_attr.reshape(E // 2, 2 * R)
    projp = _proj_tc_packed(ea32, w2a, w2b, b.reshape(1, H))
    return _sc_combine(src, dst, projp, x)
